# Initial kernel scaffold; baseline (speedup 1.0000x reference)
import functools

import jax
import jax.numpy as jnp
from jax import lax
from jax.experimental import pallas as pl
from jax.experimental.pallas import tpu as pltpu
from jax.experimental.pallas import tpu_sc as plsc

_NC = 2
_NS = 16
_NW = _NC * _NS

_mesh = plsc.VectorSubcoreMesh(
    core_axis_name="c", subcore_axis_name="s", num_cores=_NC, num_subcores=_NS
)


def _sc_gather(h, idx):
    tot = idx.shape[0]
    d = h.shape[1]
    per_w = tot // _NW
    b = 80
    steps = per_w // b

    @functools.partial(
        pl.kernel,
        out_type=jax.ShapeDtypeStruct((tot, d), jnp.float32),
        mesh=_mesh,
        scratch_types=[
            pltpu.VMEM((b,), jnp.int32),
            pltpu.VMEM((b, d), jnp.float32),
            pltpu.SemaphoreType.DMA,
        ],
    )
    def k(h_hbm, idx_hbm, out_hbm, idx_v, rows_v, sem):
        wid = lax.axis_index("s") * _NC + lax.axis_index("c")
        base0 = wid * per_w

        @pl.loop(0, steps)
        def _(i):
            base = base0 + i * b
            pltpu.sync_copy(idx_hbm.at[pl.ds(base, b)], idx_v)
            pltpu.async_copy(h_hbm.at[idx_v], rows_v, sem).wait()
            pltpu.sync_copy(rows_v, out_hbm.at[pl.ds(base, b)])

    return k(h, idx)


def _sc_scatter(ga, gb, dst, za, zb):
    e = ga.shape[0]
    n = za.shape[0]
    per_w = e // _NS
    b = 80
    steps = per_w // b

    @functools.partial(
        pl.kernel,
        out_type=(
            jax.ShapeDtypeStruct((n, 64), jnp.float32),
            jax.ShapeDtypeStruct((n, 80), jnp.float32),
        ),
        mesh=_mesh,
        scratch_types=[
            pltpu.VMEM((b,), jnp.int32),
            pltpu.VMEM((b, 64), jnp.float32),
            pltpu.VMEM((b, 80), jnp.float32),
            pltpu.VMEM_SHARED((n, 64), jnp.float32),
            pltpu.VMEM_SHARED((n, 80), jnp.float32),
        ],
    )
    def k(ga_hbm, gb_hbm, dst_hbm, za_hbm, zb_hbm, sa_hbm, sb_hbm,
          idx_v, ra_v, rb_v, acc_a, acc_b):
        c = lax.axis_index("c")
        s = lax.axis_index("s")

        @pl.when(s == 0)
        def _():
            @pl.when(c == 0)
            def _():
                pltpu.sync_copy(za_hbm, acc_a)

            @pl.when(c == 1)
            def _():
                pltpu.sync_copy(zb_hbm, acc_b)

        plsc.subcore_barrier()
        base0 = s * per_w

        @pl.loop(0, steps)
        def _(i):
            base = base0 + i * b
            pltpu.sync_copy(dst_hbm.at[pl.ds(base, b)], idx_v)

            @pl.when(c == 0)
            def _():
                pltpu.sync_copy(ga_hbm.at[pl.ds(base, b)], ra_v)
                pltpu.sync_copy(ra_v, acc_a.at[idx_v], add=True)

            @pl.when(c == 1)
            def _():
                pltpu.sync_copy(gb_hbm.at[pl.ds(base, b)], rb_v)
                pltpu.sync_copy(rb_v, acc_b.at[idx_v], add=True)

        plsc.subcore_barrier()

        @pl.when(s == 0)
        def _():
            @pl.when(c == 0)
            def _():
                pltpu.sync_copy(acc_a, sa_hbm)

            @pl.when(c == 1)
            def _():
                pltpu.sync_copy(acc_b, sb_hbm)

    return k(ga, gb, dst, za, zb)


def _gelu(x):
    return jax.nn.gelu(x, approximate=False)


def _edge_body(hs_ref, hd_ref, ef_ref, w1a_ref, w1b_ref, w1c_ref, b1_ref,
               w2_ref, ga_ref, gb_ref):
    pre = (
        jnp.dot(hs_ref[...], w1a_ref[...], preferred_element_type=jnp.float32)
        + jnp.dot(hd_ref[...], w1b_ref[...], preferred_element_type=jnp.float32)
        + jnp.dot(ef_ref[...], w1c_ref[...], preferred_element_type=jnp.float32)
        + b1_ref[...]
    )
    m = jnp.dot(_gelu(pre), w2_ref[...], preferred_element_type=jnp.float32)
    ga_ref[...] = m[:, :64]
    be = m.shape[0]
    ones_col = (lax.broadcasted_iota(jnp.int32, (be, 16), 1) == 0)
    gb_ref[...] = jnp.concatenate(
        [m[:, 64:], ones_col.astype(jnp.float32)], axis=1
    )


def _tc_edge(hg, ef, w1a, w1b, w1c, b1, w2):
    e = ef.shape[0]
    be = 512
    nb = e // be

    return pl.pallas_call(
        _edge_body,
        grid=(nb,),
        in_specs=[
            pl.BlockSpec((be, 128), lambda i: (i, 0)),
            pl.BlockSpec((be, 128), lambda i, _nb=nb: (i + _nb, 0)),
            pl.BlockSpec((be, 16), lambda i: (i, 0)),
            pl.BlockSpec((128, 256), lambda i: (0, 0)),
            pl.BlockSpec((128, 256), lambda i: (0, 0)),
            pl.BlockSpec((16, 256), lambda i: (0, 0)),
            pl.BlockSpec((1, 256), lambda i: (0, 0)),
            pl.BlockSpec((256, 128), lambda i: (0, 0)),
        ],
        out_specs=[
            pl.BlockSpec((be, 64), lambda i: (i, 0)),
            pl.BlockSpec((be, 80), lambda i: (i, 0)),
        ],
        out_shape=[
            jax.ShapeDtypeStruct((e, 64), jnp.float32),
            jax.ShapeDtypeStruct((e, 80), jnp.float32),
        ],
        compiler_params=pltpu.CompilerParams(
            dimension_semantics=("parallel",),
        ),
    )(hg, hg, ef, w1a, w1b, w1c, b1, w2)


def _node_body(h_ref, sa_ref, sb_ref, u1a_ref, u1b_ref, c1_ref, u2_ref,
               c2_ref, b2_ref, gamma_ref, beta_ref, out_ref):
    h = h_ref[...]
    sm = jnp.concatenate([sa_ref[...], sb_ref[...][:, :64]], axis=1)
    cnt = sb_ref[...][:, 64:65]
    denom = jnp.maximum(cnt, 1.0)
    gate = (cnt > 0).astype(jnp.float32)
    agg = sm / denom + b2_ref[...] * gate
    x2 = (
        jnp.dot(h, u1a_ref[...], preferred_element_type=jnp.float32)
        + jnp.dot(agg, u1b_ref[...], preferred_element_type=jnp.float32)
        + c1_ref[...]
    )
    u = jnp.dot(_gelu(x2), u2_ref[...], preferred_element_type=jnp.float32)
    x = u + c2_ref[...] + h
    mu = jnp.mean(x, axis=1, keepdims=True)
    var = jnp.mean((x - mu) ** 2, axis=1, keepdims=True)
    out_ref[...] = (x - mu) / jnp.sqrt(var + 1e-5) * gamma_ref[...] + beta_ref[...]


def _tc_node(h, sa, sb, u1a, u1b, c1, u2, c2, b2, gamma, beta):
    n = h.shape[0]
    bn = 400
    nb = n // bn

    return pl.pallas_call(
        _node_body,
        grid=(nb,),
        in_specs=[
            pl.BlockSpec((bn, 128), lambda i: (i, 0)),
            pl.BlockSpec((bn, 64), lambda i: (i, 0)),
            pl.BlockSpec((bn, 80), lambda i: (i, 0)),
            pl.BlockSpec((128, 256), lambda i: (0, 0)),
            pl.BlockSpec((128, 256), lambda i: (0, 0)),
            pl.BlockSpec((1, 256), lambda i: (0, 0)),
            pl.BlockSpec((256, 128), lambda i: (0, 0)),
            pl.BlockSpec((1, 128), lambda i: (0, 0)),
            pl.BlockSpec((1, 128), lambda i: (0, 0)),
            pl.BlockSpec((1, 128), lambda i: (0, 0)),
            pl.BlockSpec((1, 128), lambda i: (0, 0)),
        ],
        out_specs=pl.BlockSpec((bn, 128), lambda i: (i, 0)),
        out_shape=jax.ShapeDtypeStruct((n, 128), jnp.float32),
        compiler_params=pltpu.CompilerParams(
            dimension_semantics=("parallel",),
        ),
    )(h, sa, sb, u1a, u1b, c1, u2, c2, b2, gamma, beta)


def kernel(node_features, edge_index, edge_features,
           W1, b1, W2, b2, U1, c1, U2, c2, gamma, beta):
    n, d = node_features.shape
    src = edge_index[0]
    dst = edge_index[1]

    w1a = W1[:d]
    w1b = W1[d:2 * d]
    w1c = W1[2 * d:]
    u1a = U1[:d]
    u1b = U1[d:]

    idx = jnp.concatenate([src, dst])
    hg = _sc_gather(node_features, idx)

    ga, gb = _tc_edge(hg, edge_features, w1a, w1b, w1c,
                      b1.reshape(1, -1), W2)

    za = jnp.zeros((n, 64), jnp.float32)
    zb = jnp.zeros((n, 80), jnp.float32)
    sa, sb = _sc_scatter(ga, gb, dst, za, zb)

    return _tc_node(node_features, sa, sb, u1a, u1b, c1.reshape(1, -1),
                    U2, c2.reshape(1, -1), b2.reshape(1, -1),
                    gamma.reshape(1, -1), beta.reshape(1, -1))


# speedup vs baseline: 2.1391x; 2.1391x over previous
import functools

import jax
import jax.numpy as jnp
from jax import lax
from jax.experimental import pallas as pl
from jax.experimental.pallas import tpu as pltpu
from jax.experimental.pallas import tpu_sc as plsc

_NC = 2
_NS = 16
_NW = _NC * _NS

_mesh = plsc.VectorSubcoreMesh(
    core_axis_name="c", subcore_axis_name="s", num_cores=_NC, num_subcores=_NS
)


def _sc_gather(h, idx):
    tot = idx.shape[0]
    d = h.shape[1]
    per_w = tot // _NW
    b = 80
    steps = per_w // b

    @functools.partial(
        pl.kernel,
        out_type=jax.ShapeDtypeStruct((tot, d), jnp.float32),
        mesh=_mesh,
        scratch_types=[
            pltpu.VMEM((b,), jnp.int32),
            pltpu.VMEM((b, d), jnp.float32),
            pltpu.SemaphoreType.DMA,
        ],
    )
    def k(h_hbm, idx_hbm, out_hbm, idx_v, rows_v, sem):
        wid = lax.axis_index("s") * _NC + lax.axis_index("c")
        base0 = wid * per_w

        @pl.loop(0, steps)
        def _(i):
            base = base0 + i * b
            pltpu.sync_copy(idx_hbm.at[pl.ds(base, b)], idx_v)
            pltpu.async_copy(h_hbm.at[idx_v], rows_v, sem).wait()
            pltpu.sync_copy(rows_v, out_hbm.at[pl.ds(base, b)])

    return k(h, idx)


def _sc_scatter(m, dst, ones_rows, zn):
    e = m.shape[0]
    n = zn.shape[0]
    per_w = e // _NS
    b = 80
    steps = per_w // b

    @functools.partial(
        pl.kernel,
        out_type=(
            jax.ShapeDtypeStruct((n, 128), jnp.float32),
            jax.ShapeDtypeStruct((n, 128), jnp.float32),
        ),
        mesh=_mesh,
        scratch_types=[
            pltpu.VMEM((b,), jnp.int32),
            pltpu.VMEM((b, 128), jnp.float32),
            pltpu.VMEM_SHARED((n, 128), jnp.float32),
        ],
    )
    def k(m_hbm, dst_hbm, ones_hbm, zn_hbm, sm_hbm, scnt_hbm,
          idx_v, rows_v, acc):
        c = lax.axis_index("c")
        s = lax.axis_index("s")

        @pl.when(s == 0)
        def _():
            pltpu.sync_copy(zn_hbm, acc)

        @pl.when(c == 1)
        def _():
            pltpu.sync_copy(ones_hbm, rows_v)

        plsc.subcore_barrier()
        base0 = s * per_w

        @pl.loop(0, steps)
        def _(i):
            base = base0 + i * b
            pltpu.sync_copy(dst_hbm.at[pl.ds(base, b)], idx_v)

            @pl.when(c == 0)
            def _():
                pltpu.sync_copy(m_hbm.at[pl.ds(base, b)], rows_v)

            pltpu.sync_copy(rows_v, acc.at[idx_v], add=True)

        plsc.subcore_barrier()

        @pl.when(s == 0)
        def _():
            @pl.when(c == 0)
            def _():
                pltpu.sync_copy(acc, sm_hbm)

            @pl.when(c == 1)
            def _():
                pltpu.sync_copy(acc, scnt_hbm)

    return k(m, dst, ones_rows, zn)


def _gelu(x):
    return 0.5 * x * (1.0 + lax.erf(x * 0.7071067811865476))


def _edge_body(hs_ref, hd_ref, ef_ref, w1a_ref, w1b_ref, w1c_ref, b1_ref,
               w2_ref, m_ref):
    pre = (
        jnp.dot(hs_ref[...], w1a_ref[...], preferred_element_type=jnp.float32)
        + jnp.dot(hd_ref[...], w1b_ref[...], preferred_element_type=jnp.float32)
        + jnp.dot(ef_ref[...], w1c_ref[...], preferred_element_type=jnp.float32)
        + b1_ref[...]
    )
    m_ref[...] = jnp.dot(_gelu(pre), w2_ref[...],
                         preferred_element_type=jnp.float32)


def _tc_edge(hg, ef, w1a, w1b, w1c, b1, w2):
    e = ef.shape[0]
    be = 512
    nb = e // be

    return pl.pallas_call(
        _edge_body,
        grid=(nb,),
        in_specs=[
            pl.BlockSpec((be, 128), lambda i: (i, 0)),
            pl.BlockSpec((be, 128), lambda i, _nb=nb: (i + _nb, 0)),
            pl.BlockSpec((be, 16), lambda i: (i, 0)),
            pl.BlockSpec((128, 256), lambda i: (0, 0)),
            pl.BlockSpec((128, 256), lambda i: (0, 0)),
            pl.BlockSpec((16, 256), lambda i: (0, 0)),
            pl.BlockSpec((1, 256), lambda i: (0, 0)),
            pl.BlockSpec((256, 128), lambda i: (0, 0)),
        ],
        out_specs=pl.BlockSpec((be, 128), lambda i: (i, 0)),
        out_shape=jax.ShapeDtypeStruct((e, 128), jnp.float32),
        compiler_params=pltpu.CompilerParams(
            dimension_semantics=("parallel",),
        ),
    )(hg, hg, ef, w1a, w1b, w1c, b1, w2)


def _node_body(h_ref, sm_ref, scnt_ref, u1a_ref, u1b_ref, c1_ref, u2_ref,
               c2_ref, b2_ref, gamma_ref, beta_ref, out_ref):
    h = h_ref[...]
    sm = sm_ref[...]
    cnt = scnt_ref[...][:, 0:1]
    denom = jnp.maximum(cnt, 1.0)
    gate = (cnt > 0).astype(jnp.float32)
    agg = sm / denom + b2_ref[...] * gate
    x2 = (
        jnp.dot(h, u1a_ref[...], preferred_element_type=jnp.float32)
        + jnp.dot(agg, u1b_ref[...], preferred_element_type=jnp.float32)
        + c1_ref[...]
    )
    u = jnp.dot(_gelu(x2), u2_ref[...], preferred_element_type=jnp.float32)
    x = u + c2_ref[...] + h
    mu = jnp.mean(x, axis=1, keepdims=True)
    var = jnp.mean((x - mu) ** 2, axis=1, keepdims=True)
    out_ref[...] = (x - mu) / jnp.sqrt(var + 1e-5) * gamma_ref[...] + beta_ref[...]


def _tc_node(h, sm, scnt, u1a, u1b, c1, u2, c2, b2, gamma, beta):
    n = h.shape[0]
    bn = 400
    nb = n // bn

    return pl.pallas_call(
        _node_body,
        grid=(nb,),
        in_specs=[
            pl.BlockSpec((bn, 128), lambda i: (i, 0)),
            pl.BlockSpec((bn, 128), lambda i: (i, 0)),
            pl.BlockSpec((bn, 128), lambda i: (i, 0)),
            pl.BlockSpec((128, 256), lambda i: (0, 0)),
            pl.BlockSpec((128, 256), lambda i: (0, 0)),
            pl.BlockSpec((1, 256), lambda i: (0, 0)),
            pl.BlockSpec((256, 128), lambda i: (0, 0)),
            pl.BlockSpec((1, 128), lambda i: (0, 0)),
            pl.BlockSpec((1, 128), lambda i: (0, 0)),
            pl.BlockSpec((1, 128), lambda i: (0, 0)),
            pl.BlockSpec((1, 128), lambda i: (0, 0)),
        ],
        out_specs=pl.BlockSpec((bn, 128), lambda i: (i, 0)),
        out_shape=jax.ShapeDtypeStruct((n, 128), jnp.float32),
        compiler_params=pltpu.CompilerParams(
            dimension_semantics=("parallel",),
        ),
    )(h, sm, scnt, u1a, u1b, c1, u2, c2, b2, gamma, beta)


def kernel(node_features, edge_index, edge_features,
           W1, b1, W2, b2, U1, c1, U2, c2, gamma, beta):
    n, d = node_features.shape
    src = edge_index[0]
    dst = edge_index[1]

    w1a = W1[:d]
    w1b = W1[d:2 * d]
    w1c = W1[2 * d:]
    u1a = U1[:d]
    u1b = U1[d:]

    idx = jnp.concatenate([src, dst])
    hg = _sc_gather(node_features, idx)

    m = _tc_edge(hg, edge_features, w1a, w1b, w1c,
                 b1.reshape(1, -1), W2)

    ones_rows = jnp.ones((80, 128), jnp.float32)
    zn = jnp.zeros((n, 128), jnp.float32)
    sm, scnt = _sc_scatter(m, dst, ones_rows, zn)

    return _tc_node(node_features, sm, scnt, u1a, u1b, c1.reshape(1, -1),
                    U2, c2.reshape(1, -1), b2.reshape(1, -1),
                    gamma.reshape(1, -1), beta.reshape(1, -1))


# speedup vs baseline: 3.1758x; 1.4846x over previous
import functools

import jax
import jax.numpy as jnp
from jax import lax
from jax.experimental import pallas as pl
from jax.experimental.pallas import tpu as pltpu
from jax.experimental.pallas import tpu_sc as plsc

_NC = 2
_NS = 16
_NW = _NC * _NS
_B = 80

_mesh = plsc.VectorSubcoreMesh(
    core_axis_name="c", subcore_axis_name="s", num_cores=_NC, num_subcores=_NS
)


def _sc_gather(h, idx):
    tot = idx.shape[0]
    d = h.shape[1]
    per_w = tot // _NW
    steps = per_w // _B

    @functools.partial(
        pl.kernel,
        out_type=jax.ShapeDtypeStruct((tot, d), jnp.float32),
        mesh=_mesh,
        scratch_types=[
            pltpu.VMEM((_B,), jnp.int32),
            pltpu.VMEM((_B, d), jnp.float32),
            pltpu.SemaphoreType.DMA,
        ],
    )
    def k(h_hbm, idx_hbm, out_hbm, idx_v, rows_v, sem):
        wid = lax.axis_index("s") * _NC + lax.axis_index("c")
        base0 = wid * per_w

        @pl.loop(0, steps)
        def _(i):
            base = base0 + i * _B
            pltpu.sync_copy(idx_hbm.at[pl.ds(base, _B)], idx_v)
            pltpu.async_copy(h_hbm.at[idx_v], rows_v, sem).wait()
            pltpu.sync_copy(rows_v, out_hbm.at[pl.ds(base, _B)])

    return k(h, idx)


def _sc_scatter_m(m, dst, zn):
    e = m.shape[0]
    n = zn.shape[0]
    per_c = e // _NC
    per_w = per_c // _NS
    steps = per_w // _B

    @functools.partial(
        pl.kernel,
        out_type=jax.ShapeDtypeStruct((_NC, n, 128), jnp.float32),
        mesh=_mesh,
        scratch_types=[
            pltpu.VMEM((_B,), jnp.int32),
            pltpu.VMEM((_B, 128), jnp.float32),
            pltpu.VMEM_SHARED((n, 128), jnp.float32),
        ],
    )
    def k(m_hbm, dst_hbm, zn_hbm, out_hbm, idx_v, rows_v, acc):
        c = lax.axis_index("c")
        s = lax.axis_index("s")

        @pl.when(s == 0)
        def _():
            pltpu.sync_copy(zn_hbm, acc)

        plsc.subcore_barrier()
        base0 = c * per_c + s * per_w

        @pl.loop(0, steps)
        def _(i):
            base = base0 + i * _B
            pltpu.sync_copy(dst_hbm.at[pl.ds(base, _B)], idx_v)
            pltpu.sync_copy(m_hbm.at[pl.ds(base, _B)], rows_v)
            pltpu.sync_copy(rows_v, acc.at[idx_v], add=True)

        plsc.subcore_barrier()

        @pl.when(s == 0)
        def _():
            pltpu.sync_copy(acc, out_hbm.at[c])

    return k(m, dst, zn)


def _sc_cnt(dst, ones_rows, zn):
    e = dst.shape[0]
    n = zn.shape[0]
    per_c = e // _NC
    per_w = per_c // _NS
    steps = per_w // _B

    @functools.partial(
        pl.kernel,
        out_type=jax.ShapeDtypeStruct((_NC, n, 128), jnp.float32),
        mesh=_mesh,
        scratch_types=[
            pltpu.VMEM((_B,), jnp.int32),
            pltpu.VMEM((_B, 128), jnp.float32),
            pltpu.VMEM_SHARED((n, 128), jnp.float32),
        ],
    )
    def k(dst_hbm, ones_hbm, zn_hbm, out_hbm, idx_v, ones_v, acc):
        c = lax.axis_index("c")
        s = lax.axis_index("s")

        @pl.when(s == 0)
        def _():
            pltpu.sync_copy(zn_hbm, acc)

        pltpu.sync_copy(ones_hbm, ones_v)
        plsc.subcore_barrier()
        base0 = c * per_c + s * per_w

        @pl.loop(0, steps)
        def _(i):
            base = base0 + i * _B
            pltpu.sync_copy(dst_hbm.at[pl.ds(base, _B)], idx_v)
            pltpu.sync_copy(ones_v, acc.at[idx_v], add=True)

        plsc.subcore_barrier()

        @pl.when(s == 0)
        def _():
            pltpu.sync_copy(acc, out_hbm.at[c])

    return k(dst, ones_rows, zn)


def _gelu(x):
    return 0.5 * x * (1.0 + lax.erf(x * 0.7071067811865476))


def _edge_body(hs_ref, hd_ref, ef_ref, w1a_ref, w1b_ref, w1c_ref, b1_ref,
               w2_ref, m_ref):
    pre = (
        jnp.dot(hs_ref[...], w1a_ref[...], preferred_element_type=jnp.float32)
        + jnp.dot(hd_ref[...], w1b_ref[...], preferred_element_type=jnp.float32)
        + jnp.dot(ef_ref[...], w1c_ref[...], preferred_element_type=jnp.float32)
        + b1_ref[...]
    )
    m_ref[...] = jnp.dot(_gelu(pre), w2_ref[...],
                         preferred_element_type=jnp.float32)


def _tc_edge(hg, ef, w1a, w1b, w1c, b1, w2):
    e = ef.shape[0]
    be = 512
    nb = e // be

    return pl.pallas_call(
        _edge_body,
        grid=(nb,),
        in_specs=[
            pl.BlockSpec((be, 128), lambda i: (i, 0)),
            pl.BlockSpec((be, 128), lambda i, _nb=nb: (i + _nb, 0)),
            pl.BlockSpec((be, 16), lambda i: (i, 0)),
            pl.BlockSpec((128, 256), lambda i: (0, 0)),
            pl.BlockSpec((128, 256), lambda i: (0, 0)),
            pl.BlockSpec((16, 256), lambda i: (0, 0)),
            pl.BlockSpec((1, 256), lambda i: (0, 0)),
            pl.BlockSpec((256, 128), lambda i: (0, 0)),
        ],
        out_specs=pl.BlockSpec((be, 128), lambda i: (i, 0)),
        out_shape=jax.ShapeDtypeStruct((e, 128), jnp.float32),
        compiler_params=pltpu.CompilerParams(
            dimension_semantics=("parallel",),
        ),
    )(hg, hg, ef, w1a, w1b, w1c, b1, w2)


def _node_body(h_ref, p0_ref, p1_ref, p2_ref, p3_ref, p4_ref, cnt_ref,
               u1a_ref, u1b_ref, c1_ref, u2_ref, c2_ref, b2_ref,
               gamma_ref, beta_ref, out_ref):
    h = h_ref[...]
    sm = (p0_ref[0] + p0_ref[1] + p1_ref[0] + p1_ref[1]
          + p2_ref[0] + p2_ref[1] + p3_ref[0] + p3_ref[1]
          + p4_ref[0] + p4_ref[1])
    cnt = (cnt_ref[0][:, 0:1] + cnt_ref[1][:, 0:1])
    denom = jnp.maximum(cnt, 1.0)
    gate = (cnt > 0).astype(jnp.float32)
    agg = sm / denom + b2_ref[...] * gate
    x2 = (
        jnp.dot(h, u1a_ref[...], preferred_element_type=jnp.float32)
        + jnp.dot(agg, u1b_ref[...], preferred_element_type=jnp.float32)
        + c1_ref[...]
    )
    u = jnp.dot(_gelu(x2), u2_ref[...], preferred_element_type=jnp.float32)
    x = u + c2_ref[...] + h
    mu = jnp.mean(x, axis=1, keepdims=True)
    var = jnp.mean((x - mu) ** 2, axis=1, keepdims=True)
    out_ref[...] = (x - mu) / jnp.sqrt(var + 1e-5) * gamma_ref[...] + beta_ref[...]


def _tc_node(h, parts, cnt2, u1a, u1b, c1, u2, c2, b2, gamma, beta):
    n = h.shape[0]
    bn = 400
    nb = n // bn

    def blk2(i):
        return (0, i, 0)

    def blk(i):
        return (i, 0)

    def full(i):
        return (0, 0)

    return pl.pallas_call(
        _node_body,
        grid=(nb,),
        in_specs=[
            pl.BlockSpec((bn, 128), blk),
            pl.BlockSpec((_NC, bn, 128), blk2),
            pl.BlockSpec((_NC, bn, 128), blk2),
            pl.BlockSpec((_NC, bn, 128), blk2),
            pl.BlockSpec((_NC, bn, 128), blk2),
            pl.BlockSpec((_NC, bn, 128), blk2),
            pl.BlockSpec((_NC, bn, 128), blk2),
            pl.BlockSpec((128, 256), full),
            pl.BlockSpec((128, 256), full),
            pl.BlockSpec((1, 256), full),
            pl.BlockSpec((256, 128), full),
            pl.BlockSpec((1, 128), full),
            pl.BlockSpec((1, 128), full),
            pl.BlockSpec((1, 128), full),
            pl.BlockSpec((1, 128), full),
        ],
        out_specs=pl.BlockSpec((bn, 128), blk),
        out_shape=jax.ShapeDtypeStruct((n, 128), jnp.float32),
        compiler_params=pltpu.CompilerParams(
            dimension_semantics=("parallel",),
        ),
    )(h, *parts, cnt2, u1a, u1b, c1, u2, c2, b2, gamma, beta)


def kernel(node_features, edge_index, edge_features,
           W1, b1, W2, b2, U1, c1, U2, c2, gamma, beta):
    n, d = node_features.shape
    src = edge_index[0]
    dst = edge_index[1]
    e = src.shape[0]

    w1a = W1[:d]
    w1b = W1[d:2 * d]
    w1c = W1[2 * d:]
    u1a = U1[:d]
    u1b = U1[d:]
    b1r = b1.reshape(1, -1)

    nchunks = 5
    ec = e // nchunks
    idx = jnp.concatenate(
        [src.reshape(nchunks, ec), dst.reshape(nchunks, ec)], axis=1
    ).reshape(-1)

    ones_rows = jnp.ones((_B, 128), jnp.float32)
    zn = jnp.zeros((n, 128), jnp.float32)

    cnt2 = _sc_cnt(dst, ones_rows, zn)

    parts = []
    for k in range(nchunks):
        idx_k = idx[k * 2 * ec:(k + 1) * 2 * ec]
        hg_k = _sc_gather(node_features, idx_k)
        m_k = _tc_edge(hg_k, edge_features[k * ec:(k + 1) * ec],
                       w1a, w1b, w1c, b1r, W2)
        parts.append(_sc_scatter_m(m_k, dst[k * ec:(k + 1) * ec], zn))

    return _tc_node(node_features, parts, cnt2, u1a, u1b, c1.reshape(1, -1),
                    U2, c2.reshape(1, -1), b2.reshape(1, -1),
                    gamma.reshape(1, -1), beta.reshape(1, -1))


# speedup vs baseline: 3.9503x; 1.2439x over previous
import functools

import jax
import jax.numpy as jnp
from jax import lax
from jax.experimental import pallas as pl
from jax.experimental.pallas import tpu as pltpu
from jax.experimental.pallas import tpu_sc as plsc

_NC = 2
_NS = 16
_NW = _NC * _NS
_B = 80

_mesh = plsc.VectorSubcoreMesh(
    core_axis_name="c", subcore_axis_name="s", num_cores=_NC, num_subcores=_NS
)


def _split_rows(n):
    a = ((n // _NS) // 8) * 8
    return a, n - (_NS - 1) * a


def _part_copy(src, dst, s, n, dst_off=0):
    a, tail = _split_rows(n)

    @pl.when(s < _NS - 1)
    def _():
        pltpu.sync_copy(src.at[pl.ds(s * a, a)],
                        dst.at[pl.ds(dst_off + s * a, a)])

    @pl.when(s == _NS - 1)
    def _():
        pltpu.sync_copy(src.at[pl.ds((_NS - 1) * a, tail)],
                        dst.at[pl.ds(dst_off + (_NS - 1) * a, tail)])


def _sc_gather(h, idx):
    tot = idx.shape[0]
    n, d = h.shape
    per_w = tot // _NW
    steps = per_w // _B
    pairs = steps // 2

    @functools.partial(
        pl.kernel,
        out_type=jax.ShapeDtypeStruct((tot, d), jnp.float32),
        mesh=_mesh,
        scratch_types=[
            pltpu.VMEM((per_w,), jnp.int32),
            pltpu.VMEM((_B, d), jnp.float32),
            pltpu.VMEM((_B, d), jnp.float32),
            pltpu.VMEM_SHARED((n, d), jnp.float32),
            pltpu.SemaphoreType.DMA,
            pltpu.SemaphoreType.DMA,
            pltpu.SemaphoreType.DMA,
            pltpu.SemaphoreType.DMA,
        ],
    )
    def k(h_hbm, idx_hbm, out_hbm, idx_all, r0, r1, h_sp, sg0, sg1, so0, so1):
        c = lax.axis_index("c")
        s = lax.axis_index("s")
        wid = s * _NC + c
        _part_copy(h_hbm, h_sp, s, n)
        base0 = wid * per_w
        pltpu.sync_copy(idx_hbm.at[pl.ds(base0, per_w)], idx_all)
        plsc.subcore_barrier()

        def gath(step, buf, sem):
            pltpu.async_copy(
                h_sp.at[idx_all.at[pl.ds(step * _B, _B)]], buf, sem)

        def gath_wait(buf, sem):
            pltpu.make_async_copy(
                h_sp.at[idx_all.at[pl.ds(0, _B)]], buf, sem).wait()

        def wb(step, buf, sem):
            pltpu.async_copy(buf, out_hbm.at[pl.ds(base0 + step * _B, _B)],
                             sem)

        def wb_wait(buf, sem):
            pltpu.make_async_copy(
                buf, out_hbm.at[pl.ds(base0, _B)], sem).wait()

        gath(0, r0, sg0)

        @pl.loop(0, pairs)
        def _(i):
            @pl.when(i > 0)
            def _():
                wb_wait(r1, so1)

            gath(2 * i + 1, r1, sg1)
            gath_wait(r0, sg0)
            wb(2 * i, r0, so0)
            gath_wait(r1, sg1)
            wb(2 * i + 1, r1, so1)

            @pl.when(i < pairs - 1)
            def _():
                wb_wait(r0, so0)
                gath(2 * i + 2, r0, sg0)

        wb_wait(r0, so0)
        wb_wait(r1, so1)

    return k(h, idx)


def _sc_scatter_m(m, dst2d, zn, chunk_row0):
    e = m.shape[0]
    n = zn.shape[0]
    per_c = e // _NC
    per_w = per_c // _NS
    steps = per_w // _B
    pairs = steps // 2

    idx_rows = ((steps + 7) // 8) * 8 + 8

    @functools.partial(
        pl.kernel,
        out_type=jax.ShapeDtypeStruct((_NC * n, 128), jnp.float32),
        mesh=_mesh,
        scratch_types=[
            pltpu.VMEM((idx_rows, _B), jnp.int32),
            pltpu.VMEM((_B, 128), jnp.float32),
            pltpu.VMEM((_B, 128), jnp.float32),
            pltpu.VMEM_SHARED((n, 128), jnp.float32),
            pltpu.SemaphoreType.DMA,
            pltpu.SemaphoreType.DMA,
        ],
    )
    def k(m_hbm, dst2_hbm, zn_hbm, out_hbm, idx2, r0, r1, acc, sl0, sl1):
        c = lax.axis_index("c")
        s = lax.axis_index("s")
        _part_copy(zn_hbm, acc, s, n)
        row0 = chunk_row0 + c * (per_c // _B) + s * steps
        aligned0 = pl.multiple_of((row0 // 8) * 8, 8)
        delta = row0 - aligned0
        pltpu.sync_copy(dst2_hbm.at[pl.ds(aligned0, idx_rows)], idx2)
        plsc.subcore_barrier()
        base0 = c * per_c + s * per_w

        def ld(j, buf, sem):
            pltpu.async_copy(m_hbm.at[pl.ds(base0 + j * _B, _B)], buf, sem)

        def ld_wait(buf, sem):
            pltpu.make_async_copy(
                m_hbm.at[pl.ds(base0, _B)], buf, sem).wait()

        def sc(j, buf):
            pltpu.sync_copy(buf, acc.at[idx2.at[delta + j]], add=True)

        ld(0, r0, sl0)

        @pl.loop(0, pairs)
        def _(i):
            ld(2 * i + 1, r1, sl1)
            ld_wait(r0, sl0)
            sc(2 * i, r0)
            ld(2 * i + 2, r0, sl0)
            ld_wait(r1, sl1)
            sc(2 * i + 1, r1)

        ld_wait(r0, sl0)
        sc(steps - 1, r0)
        plsc.subcore_barrier()
        _part_copy(acc, out_hbm, s, n, dst_off=c * n)

    return k(m, dst2d, zn)


def _sc_cnt(dst2d, ones_rows, zn, rows_real):
    n = zn.shape[0]
    per_c_rows = rows_real // _NC
    steps = per_c_rows // _NS
    idx_rows = ((steps + 7) // 8) * 8 + 8

    @functools.partial(
        pl.kernel,
        out_type=jax.ShapeDtypeStruct((_NC * n, 128), jnp.float32),
        mesh=_mesh,
        scratch_types=[
            pltpu.VMEM((idx_rows, _B), jnp.int32),
            pltpu.VMEM((_B, 128), jnp.float32),
            pltpu.VMEM_SHARED((n, 128), jnp.float32),
        ],
    )
    def k(dst2_hbm, ones_hbm, zn_hbm, out_hbm, idx2, ones_v, acc):
        c = lax.axis_index("c")
        s = lax.axis_index("s")
        _part_copy(zn_hbm, acc, s, n)
        pltpu.sync_copy(ones_hbm, ones_v)
        row0 = c * per_c_rows + s * steps
        aligned0 = pl.multiple_of((row0 // 8) * 8, 8)
        delta = row0 - aligned0
        pltpu.sync_copy(dst2_hbm.at[pl.ds(aligned0, idx_rows)], idx2)
        plsc.subcore_barrier()

        @pl.loop(0, steps)
        def _(j):
            pltpu.sync_copy(ones_v, acc.at[idx2.at[delta + j]], add=True)

        plsc.subcore_barrier()
        _part_copy(acc, out_hbm, s, n, dst_off=c * n)

    return k(dst2d, ones_rows, zn)


def _gelu(x):
    return 0.5 * x * (1.0 + lax.erf(x * 0.7071067811865476))


def _edge_body(hs_ref, hd_ref, ef_ref, w1a_ref, w1b_ref, w1c_ref, b1_ref,
               w2_ref, m_ref):
    pre = (
        jnp.dot(hs_ref[...], w1a_ref[...], preferred_element_type=jnp.float32)
        + jnp.dot(hd_ref[...], w1b_ref[...], preferred_element_type=jnp.float32)
        + jnp.dot(ef_ref[...], w1c_ref[...], preferred_element_type=jnp.float32)
        + b1_ref[...]
    )
    m_ref[...] = jnp.dot(_gelu(pre), w2_ref[...],
                         preferred_element_type=jnp.float32)


def _tc_edge(hg, ef, w1a, w1b, w1c, b1, w2):
    e = ef.shape[0]
    be = 512
    nb = e // be

    return pl.pallas_call(
        _edge_body,
        grid=(nb,),
        in_specs=[
            pl.BlockSpec((be, 128), lambda i: (i, 0)),
            pl.BlockSpec((be, 128), lambda i, _nb=nb: (i + _nb, 0)),
            pl.BlockSpec((be, 16), lambda i: (i, 0)),
            pl.BlockSpec((128, 256), lambda i: (0, 0)),
            pl.BlockSpec((128, 256), lambda i: (0, 0)),
            pl.BlockSpec((16, 256), lambda i: (0, 0)),
            pl.BlockSpec((1, 256), lambda i: (0, 0)),
            pl.BlockSpec((256, 128), lambda i: (0, 0)),
        ],
        out_specs=pl.BlockSpec((be, 128), lambda i: (i, 0)),
        out_shape=jax.ShapeDtypeStruct((e, 128), jnp.float32),
        compiler_params=pltpu.CompilerParams(
            dimension_semantics=("parallel",),
        ),
    )(hg, hg, ef, w1a, w1b, w1c, b1, w2)


def _node_body(*refs):
    h_ref = refs[0]
    ps = refs[1:11]
    cs = refs[11:13]
    u1a_ref, u1b_ref, c1_ref, u2_ref, c2_ref, b2_ref, gamma_ref, beta_ref = \
        refs[13:21]
    out_ref = refs[21]

    h = h_ref[...]
    sm = ps[0][...]
    for p in ps[1:]:
        sm = sm + p[...]
    cnt = cs[0][...][:, 0:1] + cs[1][...][:, 0:1]
    denom = jnp.maximum(cnt, 1.0)
    gate = (cnt > 0).astype(jnp.float32)
    agg = sm / denom + b2_ref[...] * gate
    x2 = (
        jnp.dot(h, u1a_ref[...], preferred_element_type=jnp.float32)
        + jnp.dot(agg, u1b_ref[...], preferred_element_type=jnp.float32)
        + c1_ref[...]
    )
    u = jnp.dot(_gelu(x2), u2_ref[...], preferred_element_type=jnp.float32)
    x = u + c2_ref[...] + h
    mu = jnp.mean(x, axis=1, keepdims=True)
    var = jnp.mean((x - mu) ** 2, axis=1, keepdims=True)
    out_ref[...] = (x - mu) / jnp.sqrt(var + 1e-5) * gamma_ref[...] + beta_ref[...]


def _tc_node(h, parts, cnt2, u1a, u1b, c1, u2, c2, b2, gamma, beta):
    n = h.shape[0]
    bn = 400
    nb = n // bn

    def blk(i):
        return (i, 0)

    def blk_hi(i, _nb=nb):
        return (i + _nb, 0)

    def full(i):
        return (0, 0)

    part_specs = []
    part_args = []
    for p in list(parts) + [cnt2]:
        part_specs.append(pl.BlockSpec((bn, 128), blk))
        part_specs.append(pl.BlockSpec((bn, 128), blk_hi))
        part_args.append(p)
        part_args.append(p)

    return pl.pallas_call(
        _node_body,
        grid=(nb,),
        in_specs=[pl.BlockSpec((bn, 128), blk)] + part_specs + [
            pl.BlockSpec((128, 256), full),
            pl.BlockSpec((128, 256), full),
            pl.BlockSpec((1, 256), full),
            pl.BlockSpec((256, 128), full),
            pl.BlockSpec((1, 128), full),
            pl.BlockSpec((1, 128), full),
            pl.BlockSpec((1, 128), full),
            pl.BlockSpec((1, 128), full),
        ],
        out_specs=pl.BlockSpec((bn, 128), blk),
        out_shape=jax.ShapeDtypeStruct((n, 128), jnp.float32),
        compiler_params=pltpu.CompilerParams(
            dimension_semantics=("parallel",),
        ),
    )(h, *part_args, u1a, u1b, c1, u2, c2, b2, gamma, beta)


def kernel(node_features, edge_index, edge_features,
           W1, b1, W2, b2, U1, c1, U2, c2, gamma, beta):
    n, d = node_features.shape
    src = edge_index[0]
    dst = edge_index[1]
    e = src.shape[0]

    w1a = W1[:d]
    w1b = W1[d:2 * d]
    w1c = W1[2 * d:]
    u1a = U1[:d]
    u1b = U1[d:]
    b1r = b1.reshape(1, -1)

    nchunks = 5
    ec = e // nchunks
    idx = jnp.concatenate(
        [src.reshape(nchunks, ec), dst.reshape(nchunks, ec)], axis=1
    ).reshape(-1)
    dst2d = jnp.concatenate(
        [dst.reshape(-1, _B), jnp.zeros((8, _B), jnp.int32)], axis=0)

    ones_rows = jnp.ones((_B, 128), jnp.float32)
    zn = jnp.zeros((n, 128), jnp.float32)

    cnt2 = _sc_cnt(dst2d, ones_rows, zn, e // _B)

    parts = []
    for k in range(nchunks):
        idx_k = idx[k * 2 * ec:(k + 1) * 2 * ec]
        hg_k = _sc_gather(node_features, idx_k)
        m_k = _tc_edge(hg_k, edge_features[k * ec:(k + 1) * ec],
                       w1a, w1b, w1c, b1r, W2)
        parts.append(_sc_scatter_m(m_k, dst2d, zn,
                                   chunk_row0=k * (ec // _B)))

    return _tc_node(node_features, parts, cnt2, u1a, u1b, c1.reshape(1, -1),
                    U2, c2.reshape(1, -1), b2.reshape(1, -1),
                    gamma.reshape(1, -1), beta.reshape(1, -1))
